# baseline (speedup 1.0000x reference)
import jax
import jax.numpy as jnp
from jax import lax
from jax.experimental import pallas as pl

NUM_LEVELS = 3
THRESHOLD = 0.1


def _ge_kernel(h_ref, weight_ref, bias_ref, wq_ref, bq_ref, wk_ref, bk_ref,
               wa_ref, ba_ref, out_ref):
    b, n, d = h_ref.shape
    bias = bias_ref[...]
    wa = wa_ref[...]
    wa_q = wa[:, :d]
    wa_k = wa[:, d:]
    const = ba_ref[0, 0]
    bq_row = bq_ref[...]
    bk_row = bk_ref[...]
    w_ext = jnp.concatenate([weight_ref[...], wq_ref[...].T, wk_ref[...].T],
                            axis=1)
    ones_col = jnp.ones((n, 1), dtype=jnp.float32)

    row = lax.broadcasted_iota(jnp.int32, (1, n, n), 1)
    col = lax.broadcasted_iota(jnp.int32, (1, n, n), 2)
    offdiag = row != col

    hs = [h_ref[i] for i in range(b)]
    for _ in range(NUM_LEVELS):
        hws = [jnp.dot(h, w_ext, preferred_element_type=jnp.float32)
               for h in hs]
        xws = [hw[:, :d] for hw in hws]
        ss = jnp.stack(
            [jnp.concatenate(
                [jnp.dot(hw[:, d:2 * d] + bq_row, wa_q.T,
                         preferred_element_type=jnp.float32),
                 jnp.dot(hw[:, 2 * d:] + bk_row, wa_k.T,
                         preferred_element_type=jnp.float32)], axis=1)
             for hw in hws])
        sq_col = ss[:, :, 0:1]
        sk_col = ss[:, :, 1:2]
        sq_row = jnp.transpose(sq_col, (0, 2, 1))
        scores = sk_col + sq_row + const
        probs = jax.nn.sigmoid(scores)
        a_edge = jnp.where(offdiag & (probs > THRESHOLD), probs, 0.0)
        deg = jnp.stack([jnp.dot(a_edge[i], ones_col,
                                 preferred_element_type=jnp.float32)
                         for i in range(b)])
        dinv_col = jnp.where(deg > 0, lax.rsqrt(deg), 0.0)
        dinv_row = jnp.transpose(dinv_col, (0, 2, 1))
        a_norm = dinv_col * a_edge * dinv_row
        hs = [jnp.dot(a_norm[i], xws[i],
                      preferred_element_type=jnp.float32) + bias
              for i in range(b)]
    for i in range(b):
        out_ref[i] = hs[i]


def kernel(x, weight, bias, Wq, bq, Wk, bk, Wa, ba):
    b, d, n = x.shape[0], x.shape[1], x.shape[2]
    h = jnp.transpose(x, (0, 2, 1))
    bias2 = bias.reshape(1, d)
    bq2 = bq.reshape(1, d)
    bk2 = bk.reshape(1, d)
    ba2 = ba.reshape(1, 1)
    out = pl.pallas_call(
        _ge_kernel,
        out_shape=jax.ShapeDtypeStruct((b, n, d), jnp.float32),
    )(h, weight, bias2, Wq, bq2, Wk, bk2, Wa, ba2)
    return jnp.transpose(out, (0, 2, 1))


# speedup vs baseline: 1.0375x; 1.0375x over previous
import jax
import jax.numpy as jnp
from jax import lax
from jax.experimental import pallas as pl

NUM_LEVELS = 3
THRESHOLD = 0.1


def _ge_kernel(x_ref, weight_ref, bias_ref, wq_ref, bq_ref, wk_ref, bk_ref,
               wa_ref, ba_ref, out_ref):
    b, d, n = x_ref.shape
    bias = bias_ref[...]
    wa = wa_ref[...]
    wa_q = wa[:, :d]
    wa_k = wa[:, d:]
    const = ba_ref[0, 0]
    bq_row = bq_ref[...]
    bk_row = bk_ref[...]
    w_ext = jnp.concatenate([weight_ref[...], wq_ref[...].T, wk_ref[...].T],
                            axis=1)
    ones_col = jnp.ones((n, 1), dtype=jnp.float32)

    row = lax.broadcasted_iota(jnp.int32, (1, n, n), 1)
    col = lax.broadcasted_iota(jnp.int32, (1, n, n), 2)
    offdiag = row != col

    hs = [x_ref[i] for i in range(b)]
    first = True
    for level in range(NUM_LEVELS):
        if first:
            hws = [lax.dot_general(g, w_ext, (((0,), (0,)), ((), ())),
                                   preferred_element_type=jnp.float32)
                   for g in hs]
            first = False
        else:
            hws = [jnp.dot(h, w_ext, preferred_element_type=jnp.float32)
                   for h in hs]
        xws = [hw[:, :d] for hw in hws]
        ss = jnp.stack(
            [jnp.concatenate(
                [jnp.dot(hw[:, d:2 * d] + bq_row, wa_q.T,
                         preferred_element_type=jnp.float32),
                 jnp.dot(hw[:, 2 * d:] + bk_row, wa_k.T,
                         preferred_element_type=jnp.float32)], axis=1)
             for hw in hws])
        sq_col = ss[:, :, 0:1]
        sk_col = ss[:, :, 1:2]
        sq_row = jnp.transpose(sq_col, (0, 2, 1))
        scores = sk_col + sq_row + const
        probs = jax.nn.sigmoid(scores)
        a_edge = jnp.where(offdiag & (probs > THRESHOLD), probs, 0.0)
        deg = jnp.stack([jnp.dot(a_edge[i], ones_col,
                                 preferred_element_type=jnp.float32)
                         for i in range(b)])
        dinv_col = jnp.where(deg > 0, lax.rsqrt(deg), 0.0)
        dinv_row = jnp.transpose(dinv_col, (0, 2, 1))
        a_norm = dinv_col * a_edge * dinv_row
        hs = [jnp.dot(a_norm[i], xws[i],
                      preferred_element_type=jnp.float32) + bias
              for i in range(b)]
    for i in range(b):
        out_ref[i] = hs[i].T


def kernel(x, weight, bias, Wq, bq, Wk, bk, Wa, ba):
    b, d, n = x.shape[0], x.shape[1], x.shape[2]
    bias2 = bias.reshape(1, d)
    bq2 = bq.reshape(1, d)
    bk2 = bk.reshape(1, d)
    ba2 = ba.reshape(1, 1)
    return pl.pallas_call(
        _ge_kernel,
        out_shape=jax.ShapeDtypeStruct((b, d, n), jnp.float32),
    )(x, weight, bias2, Wq, bq2, Wk, bk2, Wa, ba2)


# speedup vs baseline: 1.1611x; 1.1191x over previous
import jax
import jax.numpy as jnp
from jax import lax
from jax.experimental import pallas as pl

NUM_LEVELS = 3
THRESHOLD = 0.1


def _ge_kernel(x_ref, weight_ref, bias_ref, wq_ref, bq_ref, wk_ref, bk_ref,
               wa_ref, ba_ref, out_ref):
    b, d, n = x_ref.shape
    bias = bias_ref[...]
    wa = wa_ref[...]
    wa_q = wa[:, :d]
    wa_k = wa[:, d:]
    const = (ba_ref[0, 0] + jnp.sum(bq_ref[...] * wa_q)
             + jnp.sum(bk_ref[...] * wa_k))
    zcol = jnp.zeros((d, 1), dtype=jnp.float32)
    w_qk = jnp.concatenate(
        [jnp.concatenate([wa_q.T, zcol], axis=1),
         jnp.concatenate([zcol, wa_k.T], axis=1)], axis=0)
    w_ext = jnp.concatenate([weight_ref[...], wq_ref[...].T, wk_ref[...].T],
                            axis=1)
    ones_col = jnp.ones((n, 1), dtype=jnp.float32)

    row = lax.broadcasted_iota(jnp.int32, (1, n, n), 1)
    col = lax.broadcasted_iota(jnp.int32, (1, n, n), 2)
    offdiag = row != col

    hs = [x_ref[i] for i in range(b)]
    first = True
    for level in range(NUM_LEVELS):
        if first:
            hws = [lax.dot_general(g, w_ext, (((0,), (0,)), ((), ())),
                                   preferred_element_type=jnp.float32)
                   for g in hs]
            first = False
        else:
            hws = [jnp.dot(h, w_ext, preferred_element_type=jnp.float32)
                   for h in hs]
        xws = [hw[:, :d] for hw in hws]
        ss = jnp.stack(
            [jnp.dot(hw[:, d:], w_qk, preferred_element_type=jnp.float32)
             for hw in hws])
        sq_col = ss[:, :, 0:1]
        sk_col = ss[:, :, 1:2]
        sq_row = jnp.transpose(sq_col, (0, 2, 1))
        scores = sk_col + sq_row + const
        probs = jax.nn.sigmoid(scores)
        a_edge = jnp.where(offdiag & (probs > THRESHOLD), probs, 0.0)
        deg = jnp.stack([jnp.dot(a_edge[i], ones_col,
                                 preferred_element_type=jnp.float32)
                         for i in range(b)])
        dinv_col = jnp.where(deg > 0, lax.rsqrt(deg), 0.0)
        a_scaled = dinv_col * a_edge
        xw_s = [dinv_col[i] * xws[i] for i in range(b)]
        if level < NUM_LEVELS - 1:
            hs = [jnp.dot(a_scaled[i], xw_s[i],
                          preferred_element_type=jnp.float32) + bias
                  for i in range(b)]
        else:
            for i in range(b):
                out_ref[i] = lax.dot_general(
                    xw_s[i], a_scaled[i], (((0,), (1,)), ((), ())),
                    preferred_element_type=jnp.float32) + bias.T


def kernel(x, weight, bias, Wq, bq, Wk, bk, Wa, ba):
    b, d, n = x.shape[0], x.shape[1], x.shape[2]
    bias2 = bias.reshape(1, d)
    bq2 = bq.reshape(1, d)
    bk2 = bk.reshape(1, d)
    ba2 = ba.reshape(1, 1)
    return pl.pallas_call(
        _ge_kernel,
        out_shape=jax.ShapeDtypeStruct((b, d, n), jnp.float32),
    )(x, weight, bias2, Wq, bq2, Wk, bk2, Wa, ba2)


# speedup vs baseline: 1.1647x; 1.0031x over previous
import jax
import jax.numpy as jnp
from jax import lax
from jax.experimental import pallas as pl

NUM_LEVELS = 3
THRESHOLD = 0.1


def _ge_kernel(x_ref, weight_ref, bias_ref, wq_ref, bq_ref, wk_ref, bk_ref,
               wa_ref, ba_ref, out_ref):
    b, d, n = x_ref.shape
    bias = bias_ref[...]
    wa = wa_ref[...]
    wa_q = wa[:, :d]
    wa_k = wa[:, d:]
    const = (ba_ref[0, 0] + jnp.sum(bq_ref[...] * wa_q)
             + jnp.sum(bk_ref[...] * wa_k))
    zcol = jnp.zeros((d, 1), dtype=jnp.float32)
    w_qk = jnp.concatenate(
        [jnp.concatenate([wa_q.T, zcol], axis=1),
         jnp.concatenate([zcol, wa_k.T], axis=1)], axis=0)
    w_ext = jnp.concatenate([weight_ref[...], wq_ref[...].T, wk_ref[...].T],
                            axis=1)
    ones_col = jnp.ones((n, 1), dtype=jnp.float32)

    row = lax.broadcasted_iota(jnp.int32, (1, n, n), 1)
    col = lax.broadcasted_iota(jnp.int32, (1, n, n), 2)
    diag_neg = jnp.where(row == col, -jnp.inf, 0.0)

    hs = [x_ref[i] for i in range(b)]
    first = True
    for level in range(NUM_LEVELS):
        if first:
            hws = [lax.dot_general(g, w_ext, (((0,), (0,)), ((), ())),
                                   preferred_element_type=jnp.float32)
                   for g in hs]
            first = False
        else:
            hws = [jnp.dot(h, w_ext, preferred_element_type=jnp.float32)
                   for h in hs]
        xws = [hw[:, :d] for hw in hws]
        ss = jnp.stack(
            [jnp.dot(hw[:, d:], w_qk, preferred_element_type=jnp.float32)
             for hw in hws])
        sq_col = ss[:, :, 0:1]
        sk_col = ss[:, :, 1:2] + const
        sq_row = jnp.transpose(sq_col, (0, 2, 1))
        scores = (sk_col + sq_row) + diag_neg
        probs = jax.nn.sigmoid(scores)
        a_edge = jnp.where(probs > THRESHOLD, probs, 0.0)
        degs = [jnp.dot(a_edge[i], ones_col,
                        preferred_element_type=jnp.float32)
                for i in range(b)]
        dinvs = [jnp.where(dg > 0, lax.rsqrt(dg), 0.0) for dg in degs]
        a_scaled = [dinvs[i] * a_edge[i] for i in range(b)]
        xw_s = [dinvs[i] * xws[i] for i in range(b)]
        if level < NUM_LEVELS - 1:
            hs = [jnp.dot(a_scaled[i], xw_s[i],
                          preferred_element_type=jnp.float32) + bias
                  for i in range(b)]
        else:
            for i in range(b):
                out_ref[i] = lax.dot_general(
                    xw_s[i], a_scaled[i], (((0,), (1,)), ((), ())),
                    preferred_element_type=jnp.float32) + bias.T


def kernel(x, weight, bias, Wq, bq, Wk, bk, Wa, ba):
    b, d, n = x.shape[0], x.shape[1], x.shape[2]
    bias2 = bias.reshape(1, d)
    bq2 = bq.reshape(1, d)
    bk2 = bk.reshape(1, d)
    ba2 = ba.reshape(1, 1)
    return pl.pallas_call(
        _ge_kernel,
        out_shape=jax.ShapeDtypeStruct((b, d, n), jnp.float32),
    )(x, weight, bias2, Wq, bq2, Wk, bk2, Wa, ba2)
